# Initial kernel scaffold; baseline (speedup 1.0000x reference)
import functools

import jax
import jax.numpy as jnp
from jax import lax
from jax.experimental import pallas as pl
from jax.experimental.pallas import tpu as pltpu
from jax.experimental.pallas import tpu_sc as plsc

B_ = 8192
D_IN_ = 512
HID_ = 64
COMM_ = 256
K_ = 8192
BETA_ = 0.25
KL_WEIGHT_ = 1.0

TB = 256
KT = 1024
NBT = B_ // TB
NKT = K_ // KT


def _fused_body(x_ref, we, be, w0, b0, w1, b1, wmu, bmu, wvar, bvar,
                pt_ref, eps_ref, closest_ref, mu_ref, kld_ref):
    x = x_ref[...]
    emb = jnp.dot(x, we[...]) + be[...]
    h = jnp.maximum(jnp.dot(emb, w0[...]) + b0[...], 0.0)
    h = jnp.maximum(jnp.dot(h, w1[...]) + b1[...], 0.0)
    mu = jnp.dot(h, wmu[...]) + bmu[...]
    lv = jnp.dot(h, wvar[...]) + bvar[...]
    sample = mu + eps_ref[...] * jnp.exp(0.5 * lv)
    mu_ref[...] = mu

    snorm = jnp.sum(sample ** 2, axis=1, keepdims=True)

    def step(k, carry):
        rmin, ridx = carry
        off = pl.multiple_of(k * KT, KT)
        pt = pt_ref[:, pl.ds(off, KT)]
        pn = jnp.sum(pt ** 2, axis=0, keepdims=True)
        m = jnp.dot(sample, pt)
        d = snorm + pn - 2.0 * m
        cmin = jnp.min(d, axis=1, keepdims=True)
        iota = lax.broadcasted_iota(jnp.int32, (TB, KT), 1) + k * KT
        cidx = jnp.min(jnp.where(d == cmin, iota, K_), axis=1, keepdims=True)
        better = cmin < rmin
        return (jnp.where(better, cmin, rmin), jnp.where(better, cidx, ridx))

    init = (jnp.full((TB, 1), jnp.inf, jnp.float32),
            jnp.zeros((TB, 1), jnp.int32))
    _, ridx = lax.fori_loop(0, NKT, step, init)
    closest_ref[...] = ridx

    i = pl.program_id(0)

    @pl.when(i == 0)
    def _():
        kld_ref[0, 0] = 0.0

    kld_ref[0, 0] += jnp.sum((1.0 + lv) - mu ** 2 - jnp.exp(lv))


def _loss_body(q_ref, mu_ref, kldsum_ref, total_ref, cap_ref):
    i = pl.program_id(0)

    @pl.when(i == 0)
    def _():
        total_ref[0, 0] = 0.0

    diff = q_ref[...] - mu_ref[...]
    total_ref[0, 0] += jnp.sum(diff * diff)

    @pl.when(i == NBT - 1)
    def _():
        msq = total_ref[0, 0] / (B_ * COMM_)
        vq = 1.0 * (msq * BETA_ + 1.0 * msq)
        kld = -0.5 * kldsum_ref[0, 0] / B_
        total_ref[0, 0] = KL_WEIGHT_ * kld + vq
        cap_ref[0, 0] = kld


_SC_INFO = plsc.get_sparse_core_info()
_NC = _SC_INFO.num_cores
_NS = _SC_INFO.num_subcores
_NW = _NC * _NS
_BPW = B_ // _NW


@functools.partial(
    pl.kernel,
    out_type=jax.ShapeDtypeStruct((B_, COMM_), jnp.float32),
    mesh=plsc.VectorSubcoreMesh(core_axis_name="c", subcore_axis_name="s"),
    scratch_types=[
        pltpu.VMEM((_BPW,), jnp.int32),
        pltpu.VMEM((_BPW, COMM_), jnp.float32),
        pltpu.SemaphoreType.DMA,
    ],
)
def _sc_gather(table_hbm, idx_hbm, out_hbm, idx_v, rows_v, sem):
    wid = lax.axis_index("s") * _NC + lax.axis_index("c")
    base = wid * _BPW
    pltpu.sync_copy(idx_hbm.at[pl.ds(base, _BPW)], idx_v)
    pltpu.async_copy(table_hbm.at[idx_v], rows_v, sem).wait()
    pltpu.sync_copy(rows_v, out_hbm.at[pl.ds(base, _BPW)])


def kernel(x, W_emb, b_emb, W0, b0, W1, b1, W_mu, b_mu, W_var, b_var,
           protos, eps):
    pt = protos.T
    b_emb2 = b_emb.reshape(1, HID_)
    b02 = b0.reshape(1, HID_)
    b12 = b1.reshape(1, COMM_)
    b_mu2 = b_mu.reshape(1, COMM_)
    b_var2 = b_var.reshape(1, COMM_)

    def full(shape):
        return pl.BlockSpec(shape, lambda i: (0, 0))

    closest2d, mu, kld_sum = pl.pallas_call(
        _fused_body,
        grid=(NBT,),
        in_specs=[
            pl.BlockSpec((TB, D_IN_), lambda i: (i, 0)),
            full((D_IN_, HID_)),
            full((1, HID_)),
            full((HID_, HID_)),
            full((1, HID_)),
            full((HID_, COMM_)),
            full((1, COMM_)),
            full((COMM_, COMM_)),
            full((1, COMM_)),
            full((COMM_, COMM_)),
            full((1, COMM_)),
            full((COMM_, K_)),
            pl.BlockSpec((TB, COMM_), lambda i: (i, 0)),
        ],
        out_specs=[
            pl.BlockSpec((TB, 1), lambda i: (i, 0)),
            pl.BlockSpec((TB, COMM_), lambda i: (i, 0)),
            pl.BlockSpec(memory_space=pltpu.SMEM),
        ],
        out_shape=[
            jax.ShapeDtypeStruct((B_, 1), jnp.int32),
            jax.ShapeDtypeStruct((B_, COMM_), jnp.float32),
            jax.ShapeDtypeStruct((1, 1), jnp.float32),
        ],
    )(x, W_emb, b_emb2, W0, b02, W1, b12, W_mu, b_mu2, W_var, b_var2,
      pt, eps)

    closest = closest2d.reshape(B_)
    quantized = _sc_gather(protos, closest)

    total, cap = pl.pallas_call(
        _loss_body,
        grid=(NBT,),
        in_specs=[
            pl.BlockSpec((TB, COMM_), lambda i: (i, 0)),
            pl.BlockSpec((TB, COMM_), lambda i: (i, 0)),
            pl.BlockSpec(memory_space=pltpu.SMEM),
        ],
        out_specs=[
            pl.BlockSpec(memory_space=pltpu.SMEM),
            pl.BlockSpec(memory_space=pltpu.SMEM),
        ],
        out_shape=[
            jax.ShapeDtypeStruct((1, 1), jnp.float32),
            jax.ShapeDtypeStruct((1, 1), jnp.float32),
        ],
    )(quantized, mu, kld_sum)

    return (quantized, total.reshape(()), cap.reshape(()))


# speedup vs baseline: 1.9257x; 1.9257x over previous
import functools

import jax
import jax.numpy as jnp
from jax import lax
from jax.experimental import pallas as pl
from jax.experimental.pallas import tpu as pltpu
from jax.experimental.pallas import tpu_sc as plsc

B_ = 8192
D_IN_ = 512
HID_ = 64
COMM_ = 256
K_ = 8192
BETA_ = 0.25
KL_WEIGHT_ = 1.0

TB = 256
KT = 1024
NBT = B_ // TB
NKT = K_ // KT


def _fused_body(x_ref, we, be, w0, b0, w1, b1, wmu, bmu, wvar, bvar,
                pt_ref, eps_ref, closest_ref, mu_ref, kld_ref):
    x = x_ref[...]
    emb = jnp.dot(x, we[...]) + be[...]
    h = jnp.maximum(jnp.dot(emb, w0[...]) + b0[...], 0.0)
    h = jnp.maximum(jnp.dot(h, w1[...]) + b1[...], 0.0)
    mu = jnp.dot(h, wmu[...]) + bmu[...]
    lv = jnp.dot(h, wvar[...]) + bvar[...]
    sample = mu + eps_ref[...] * jnp.exp(0.5 * lv)
    mu_ref[...] = mu

    snorm = jnp.sum(sample ** 2, axis=1, keepdims=True)

    def step(k, carry):
        rmin, ridx = carry
        off = pl.multiple_of(k * KT, KT)
        pt = pt_ref[:, pl.ds(off, KT)]
        pn = jnp.sum(pt ** 2, axis=0, keepdims=True)
        m = jnp.dot(sample, pt)
        d = snorm + pn - 2.0 * m
        cmin = jnp.min(d, axis=1, keepdims=True)
        iota = lax.broadcasted_iota(jnp.int32, (TB, KT), 1) + k * KT
        cidx = jnp.min(jnp.where(d == cmin, iota, K_), axis=1, keepdims=True)
        better = cmin < rmin
        return (jnp.where(better, cmin, rmin), jnp.where(better, cidx, ridx))

    init = (jnp.full((TB, 1), jnp.inf, jnp.float32),
            jnp.zeros((TB, 1), jnp.int32))
    _, ridx = lax.fori_loop(0, NKT, step, init)
    closest_ref[...] = ridx

    i = pl.program_id(0)

    @pl.when(i == 0)
    def _():
        kld_ref[0, 0] = 0.0

    kld_ref[0, 0] += jnp.sum((1.0 + lv) - mu ** 2 - jnp.exp(lv))


def _loss_body(q_ref, mu_ref, kldsum_ref, total_ref, cap_ref):
    i = pl.program_id(0)

    @pl.when(i == 0)
    def _():
        total_ref[0, 0] = 0.0

    diff = q_ref[...] - mu_ref[...]
    total_ref[0, 0] += jnp.sum(diff * diff)

    @pl.when(i == NBT - 1)
    def _():
        msq = total_ref[0, 0] / (B_ * COMM_)
        vq = 1.0 * (msq * BETA_ + 1.0 * msq)
        kld = -0.5 * kldsum_ref[0, 0] / B_
        total_ref[0, 0] = KL_WEIGHT_ * kld + vq
        cap_ref[0, 0] = kld


@functools.cache
def _make_sc_gather():
    info = plsc.get_sparse_core_info()
    nc, ns = info.num_cores, info.num_subcores
    bpw = B_ // (nc * ns)

    @functools.partial(
        pl.kernel,
        out_type=jax.ShapeDtypeStruct((B_, COMM_), jnp.float32),
        mesh=plsc.VectorSubcoreMesh(core_axis_name="c", subcore_axis_name="s"),
        scratch_types=[
            pltpu.VMEM((bpw,), jnp.int32),
            pltpu.VMEM((bpw, COMM_), jnp.float32),
            pltpu.SemaphoreType.DMA,
        ],
    )
    def _sc_gather(table_hbm, idx_hbm, out_hbm, idx_v, rows_v, sem):
        wid = lax.axis_index("s") * nc + lax.axis_index("c")
        base = wid * bpw
        pltpu.sync_copy(idx_hbm.at[pl.ds(base, bpw)], idx_v)
        pltpu.async_copy(table_hbm.at[idx_v], rows_v, sem).wait()
        pltpu.sync_copy(rows_v, out_hbm.at[pl.ds(base, bpw)])

    return _sc_gather


def kernel(x, W_emb, b_emb, W0, b0, W1, b1, W_mu, b_mu, W_var, b_var,
           protos, eps):
    pt = protos.T
    b_emb2 = b_emb.reshape(1, HID_)
    b02 = b0.reshape(1, HID_)
    b12 = b1.reshape(1, COMM_)
    b_mu2 = b_mu.reshape(1, COMM_)
    b_var2 = b_var.reshape(1, COMM_)

    def full(shape):
        return pl.BlockSpec(shape, lambda i: (0, 0))

    closest2d, mu, kld_sum = pl.pallas_call(
        _fused_body,
        grid=(NBT,),
        in_specs=[
            pl.BlockSpec((TB, D_IN_), lambda i: (i, 0)),
            full((D_IN_, HID_)),
            full((1, HID_)),
            full((HID_, HID_)),
            full((1, HID_)),
            full((HID_, COMM_)),
            full((1, COMM_)),
            full((COMM_, COMM_)),
            full((1, COMM_)),
            full((COMM_, COMM_)),
            full((1, COMM_)),
            full((COMM_, K_)),
            pl.BlockSpec((TB, COMM_), lambda i: (i, 0)),
        ],
        out_specs=[
            pl.BlockSpec((TB, 1), lambda i: (i, 0)),
            pl.BlockSpec((TB, COMM_), lambda i: (i, 0)),
            pl.BlockSpec(memory_space=pltpu.SMEM),
        ],
        out_shape=[
            jax.ShapeDtypeStruct((B_, 1), jnp.int32),
            jax.ShapeDtypeStruct((B_, COMM_), jnp.float32),
            jax.ShapeDtypeStruct((1, 1), jnp.float32),
        ],
    )(x, W_emb, b_emb2, W0, b02, W1, b12, W_mu, b_mu2, W_var, b_var2,
      pt, eps)

    closest = closest2d.reshape(B_)
    quantized = _make_sc_gather()(protos, closest)

    total, cap = pl.pallas_call(
        _loss_body,
        grid=(NBT,),
        in_specs=[
            pl.BlockSpec((TB, COMM_), lambda i: (i, 0)),
            pl.BlockSpec((TB, COMM_), lambda i: (i, 0)),
            pl.BlockSpec(memory_space=pltpu.SMEM),
        ],
        out_specs=[
            pl.BlockSpec(memory_space=pltpu.SMEM),
            pl.BlockSpec(memory_space=pltpu.SMEM),
        ],
        out_shape=[
            jax.ShapeDtypeStruct((1, 1), jnp.float32),
            jax.ShapeDtypeStruct((1, 1), jnp.float32),
        ],
    )(quantized, mu, kld_sum)

    return (quantized, total.reshape(()), cap.reshape(()))


# speedup vs baseline: 2.9374x; 1.5254x over previous
import functools

import jax
import jax.numpy as jnp
from jax import lax
from jax.experimental import pallas as pl
from jax.experimental.pallas import tpu as pltpu
from jax.experimental.pallas import tpu_sc as plsc

B_ = 8192
D_IN_ = 512
HID_ = 64
COMM_ = 256
K_ = 8192
BETA_ = 0.25
KL_WEIGHT_ = 1.0

TB = 512
KT = 1024
NBT = B_ // TB
NKT = K_ // KT


def _fused_body(x_ref, we, be, w0, b0, w1, b1, wmu, bmu, wvar, bvar,
                pt_ref, eps_ref, closest_ref, mu_ref, kld_ref, pn_ref):
    i = pl.program_id(0)

    @pl.when(i == 0)
    def _():
        kld_ref[0, 0] = 0.0
        for k in range(NKT):
            pt = pt_ref[:, k * KT:(k + 1) * KT]
            pn_ref[:, k * KT:(k + 1) * KT] = jnp.sum(
                pt ** 2, axis=0, keepdims=True)

    x = x_ref[...]
    emb = jnp.dot(x, we[...]) + be[...]
    h = jnp.maximum(jnp.dot(emb, w0[...]) + b0[...], 0.0)
    h = jnp.maximum(jnp.dot(h, w1[...]) + b1[...], 0.0)
    mu = jnp.dot(h, wmu[...]) + bmu[...]
    lv = jnp.dot(h, wvar[...]) + bvar[...]
    sample = mu + eps_ref[...] * jnp.exp(0.5 * lv)
    mu_ref[...] = mu

    snorm = jnp.sum(sample ** 2, axis=1, keepdims=True)

    rmin = jnp.full((TB, 1), jnp.inf, jnp.float32)
    ridx = jnp.zeros((TB, 1), jnp.int32)
    for k in range(NKT):
        pt = pt_ref[:, k * KT:(k + 1) * KT]
        pn = pn_ref[:, k * KT:(k + 1) * KT]
        m = jnp.dot(sample, pt)
        d = snorm + pn - 2.0 * m
        cmin = jnp.min(d, axis=1, keepdims=True)
        iota = lax.broadcasted_iota(jnp.int32, (TB, KT), 1) + k * KT
        cidx = jnp.min(jnp.where(d == cmin, iota, K_), axis=1, keepdims=True)
        better = cmin < rmin
        rmin = jnp.where(better, cmin, rmin)
        ridx = jnp.where(better, cidx, ridx)
    closest_ref[...] = ridx

    kld_ref[0, 0] += jnp.sum((1.0 + lv) - mu ** 2 - jnp.exp(lv))


def _loss_body(q_ref, mu_ref, kldsum_ref, total_ref, cap_ref):
    i = pl.program_id(0)

    @pl.when(i == 0)
    def _():
        total_ref[0, 0] = 0.0

    diff = q_ref[...] - mu_ref[...]
    total_ref[0, 0] += jnp.sum(diff * diff)

    @pl.when(i == NBT - 1)
    def _():
        msq = total_ref[0, 0] / (B_ * COMM_)
        vq = 1.0 * (msq * BETA_ + 1.0 * msq)
        kld = -0.5 * kldsum_ref[0, 0] / B_
        total_ref[0, 0] = KL_WEIGHT_ * kld + vq
        cap_ref[0, 0] = kld


@functools.cache
def _make_sc_gather():
    info = plsc.get_sparse_core_info()
    nc, ns = info.num_cores, info.num_subcores
    bpw = B_ // (nc * ns)

    @functools.partial(
        pl.kernel,
        out_type=jax.ShapeDtypeStruct((B_, COMM_), jnp.float32),
        mesh=plsc.VectorSubcoreMesh(core_axis_name="c", subcore_axis_name="s"),
        scratch_types=[
            pltpu.VMEM((bpw,), jnp.int32),
            pltpu.VMEM((bpw, COMM_), jnp.float32),
            pltpu.SemaphoreType.DMA,
        ],
    )
    def _sc_gather(table_hbm, idx_hbm, out_hbm, idx_v, rows_v, sem):
        wid = lax.axis_index("s") * nc + lax.axis_index("c")
        base = wid * bpw
        pltpu.sync_copy(idx_hbm.at[pl.ds(base, bpw)], idx_v)
        pltpu.async_copy(table_hbm.at[idx_v], rows_v, sem).wait()
        pltpu.sync_copy(rows_v, out_hbm.at[pl.ds(base, bpw)])

    return _sc_gather


def kernel(x, W_emb, b_emb, W0, b0, W1, b1, W_mu, b_mu, W_var, b_var,
           protos, eps):
    pt = protos.T
    b_emb2 = b_emb.reshape(1, HID_)
    b02 = b0.reshape(1, HID_)
    b12 = b1.reshape(1, COMM_)
    b_mu2 = b_mu.reshape(1, COMM_)
    b_var2 = b_var.reshape(1, COMM_)

    def full(shape):
        return pl.BlockSpec(shape, lambda i: (0, 0))

    closest2d, mu, kld_sum = pl.pallas_call(
        _fused_body,
        grid=(NBT,),
        in_specs=[
            pl.BlockSpec((TB, D_IN_), lambda i: (i, 0)),
            full((D_IN_, HID_)),
            full((1, HID_)),
            full((HID_, HID_)),
            full((1, HID_)),
            full((HID_, COMM_)),
            full((1, COMM_)),
            full((COMM_, COMM_)),
            full((1, COMM_)),
            full((COMM_, COMM_)),
            full((1, COMM_)),
            full((COMM_, K_)),
            pl.BlockSpec((TB, COMM_), lambda i: (i, 0)),
        ],
        out_specs=[
            pl.BlockSpec((TB, 1), lambda i: (i, 0)),
            pl.BlockSpec((TB, COMM_), lambda i: (i, 0)),
            pl.BlockSpec(memory_space=pltpu.SMEM),
        ],
        out_shape=[
            jax.ShapeDtypeStruct((B_, 1), jnp.int32),
            jax.ShapeDtypeStruct((B_, COMM_), jnp.float32),
            jax.ShapeDtypeStruct((1, 1), jnp.float32),
        ],
        scratch_shapes=[pltpu.VMEM((1, K_), jnp.float32)],
    )(x, W_emb, b_emb2, W0, b02, W1, b12, W_mu, b_mu2, W_var, b_var2,
      pt, eps)

    closest = closest2d.reshape(B_)
    quantized = _make_sc_gather()(protos, closest)

    total, cap = pl.pallas_call(
        _loss_body,
        grid=(NBT,),
        in_specs=[
            pl.BlockSpec((TB, COMM_), lambda i: (i, 0)),
            pl.BlockSpec((TB, COMM_), lambda i: (i, 0)),
            pl.BlockSpec(memory_space=pltpu.SMEM),
        ],
        out_specs=[
            pl.BlockSpec(memory_space=pltpu.SMEM),
            pl.BlockSpec(memory_space=pltpu.SMEM),
        ],
        out_shape=[
            jax.ShapeDtypeStruct((1, 1), jnp.float32),
            jax.ShapeDtypeStruct((1, 1), jnp.float32),
        ],
    )(quantized, mu, kld_sum)

    return (quantized, total.reshape(()), cap.reshape(()))


# speedup vs baseline: 3.3613x; 1.1443x over previous
import functools

import jax
import jax.numpy as jnp
from jax import lax
from jax.experimental import pallas as pl
from jax.experimental.pallas import tpu as pltpu
from jax.experimental.pallas import tpu_sc as plsc

B_ = 8192
D_IN_ = 512
HID_ = 64
COMM_ = 256
K_ = 8192
BETA_ = 0.25
KL_WEIGHT_ = 1.0

TB = 512
KT = 1024
NBT = B_ // TB
NKT = K_ // KT


def _fused_body(x_ref, we, be, w0, b0, w1, b1, wmu, bmu, wvar, bvar,
                pt_ref, eps_ref, closest_ref, mu_ref, kld_ref, pn_ref):
    i = pl.program_id(0)

    @pl.when(i == 0)
    def _():
        kld_ref[0, 0] = 0.0
        for k in range(NKT):
            pt = pt_ref[:, k * KT:(k + 1) * KT]
            pn_ref[:, k * KT:(k + 1) * KT] = jnp.sum(
                pt ** 2, axis=0, keepdims=True)

    x = x_ref[...]
    emb = jnp.dot(x, we[...]) + be[...]
    h = jnp.maximum(jnp.dot(emb, w0[...]) + b0[...], 0.0)
    h = jnp.maximum(jnp.dot(h, w1[...]) + b1[...], 0.0)
    mu = jnp.dot(h, wmu[...]) + bmu[...]
    lv = jnp.dot(h, wvar[...]) + bvar[...]
    sample = mu + eps_ref[...] * jnp.exp(0.5 * lv)
    mu_ref[...] = mu

    snorm = jnp.sum(sample ** 2, axis=1, keepdims=True)

    iotaf = lax.broadcasted_iota(jnp.int32, (1, KT), 1).astype(jnp.float32)
    rmin = jnp.full((TB, 1), jnp.inf, jnp.float32)
    ridxf = jnp.zeros((TB, 1), jnp.float32)
    for k in range(NKT):
        pt = pt_ref[:, k * KT:(k + 1) * KT]
        pn = pn_ref[:, k * KT:(k + 1) * KT]
        m = jnp.dot(sample, pt)
        d = snorm + pn - 2.0 * m
        cmin = jnp.min(d, axis=1, keepdims=True)
        cidxf = jnp.min(jnp.where(d == cmin, iotaf + (k * KT), float(K_)),
                        axis=1, keepdims=True)
        better = cmin < rmin
        rmin = jnp.where(better, cmin, rmin)
        ridxf = jnp.where(better, cidxf, ridxf)
    closest_ref[...] = ridxf.astype(jnp.int32)

    kld_ref[0, 0] += jnp.sum((1.0 + lv) - mu ** 2 - jnp.exp(lv))


def _loss_body(q_ref, mu_ref, kldsum_ref, total_ref, cap_ref):
    i = pl.program_id(0)

    @pl.when(i == 0)
    def _():
        total_ref[0, 0] = 0.0

    diff = q_ref[...] - mu_ref[...]
    total_ref[0, 0] += jnp.sum(diff * diff)

    @pl.when(i == NBT - 1)
    def _():
        msq = total_ref[0, 0] / (B_ * COMM_)
        vq = 1.0 * (msq * BETA_ + 1.0 * msq)
        kld = -0.5 * kldsum_ref[0, 0] / B_
        total_ref[0, 0] = KL_WEIGHT_ * kld + vq
        cap_ref[0, 0] = kld


@functools.cache
def _make_sc_gather():
    info = plsc.get_sparse_core_info()
    nc, ns = info.num_cores, info.num_subcores
    bpw = B_ // (nc * ns)

    @functools.partial(
        pl.kernel,
        out_type=jax.ShapeDtypeStruct((B_, COMM_), jnp.float32),
        mesh=plsc.VectorSubcoreMesh(core_axis_name="c", subcore_axis_name="s"),
        scratch_types=[
            pltpu.VMEM((bpw,), jnp.int32),
            pltpu.VMEM((bpw, COMM_), jnp.float32),
            pltpu.SemaphoreType.DMA,
        ],
    )
    def _sc_gather(table_hbm, idx_hbm, out_hbm, idx_v, rows_v, sem):
        wid = lax.axis_index("s") * nc + lax.axis_index("c")
        base = wid * bpw
        pltpu.sync_copy(idx_hbm.at[pl.ds(base, bpw)], idx_v)
        pltpu.async_copy(table_hbm.at[idx_v], rows_v, sem).wait()
        pltpu.sync_copy(rows_v, out_hbm.at[pl.ds(base, bpw)])

    return _sc_gather


def kernel(x, W_emb, b_emb, W0, b0, W1, b1, W_mu, b_mu, W_var, b_var,
           protos, eps):
    pt = protos.T
    b_emb2 = b_emb.reshape(1, HID_)
    b02 = b0.reshape(1, HID_)
    b12 = b1.reshape(1, COMM_)
    b_mu2 = b_mu.reshape(1, COMM_)
    b_var2 = b_var.reshape(1, COMM_)

    def full(shape):
        return pl.BlockSpec(shape, lambda i: (0, 0))

    closest2d, mu, kld_sum = pl.pallas_call(
        _fused_body,
        grid=(NBT,),
        in_specs=[
            pl.BlockSpec((TB, D_IN_), lambda i: (i, 0)),
            full((D_IN_, HID_)),
            full((1, HID_)),
            full((HID_, HID_)),
            full((1, HID_)),
            full((HID_, COMM_)),
            full((1, COMM_)),
            full((COMM_, COMM_)),
            full((1, COMM_)),
            full((COMM_, COMM_)),
            full((1, COMM_)),
            full((COMM_, K_)),
            pl.BlockSpec((TB, COMM_), lambda i: (i, 0)),
        ],
        out_specs=[
            pl.BlockSpec((TB, 1), lambda i: (i, 0)),
            pl.BlockSpec((TB, COMM_), lambda i: (i, 0)),
            pl.BlockSpec(memory_space=pltpu.SMEM),
        ],
        out_shape=[
            jax.ShapeDtypeStruct((B_, 1), jnp.int32),
            jax.ShapeDtypeStruct((B_, COMM_), jnp.float32),
            jax.ShapeDtypeStruct((1, 1), jnp.float32),
        ],
        scratch_shapes=[pltpu.VMEM((1, K_), jnp.float32)],
    )(x, W_emb, b_emb2, W0, b02, W1, b12, W_mu, b_mu2, W_var, b_var2,
      pt, eps)

    closest = closest2d.reshape(B_)
    quantized = _make_sc_gather()(protos, closest)

    total, cap = pl.pallas_call(
        _loss_body,
        grid=(NBT,),
        in_specs=[
            pl.BlockSpec((TB, COMM_), lambda i: (i, 0)),
            pl.BlockSpec((TB, COMM_), lambda i: (i, 0)),
            pl.BlockSpec(memory_space=pltpu.SMEM),
        ],
        out_specs=[
            pl.BlockSpec(memory_space=pltpu.SMEM),
            pl.BlockSpec(memory_space=pltpu.SMEM),
        ],
        out_shape=[
            jax.ShapeDtypeStruct((1, 1), jnp.float32),
            jax.ShapeDtypeStruct((1, 1), jnp.float32),
        ],
    )(quantized, mu, kld_sum)

    return (quantized, total.reshape(()), cap.reshape(()))


# speedup vs baseline: 3.4630x; 1.0303x over previous
import functools

import jax
import jax.numpy as jnp
from jax import lax
from jax.experimental import pallas as pl
from jax.experimental.pallas import tpu as pltpu
from jax.experimental.pallas import tpu_sc as plsc

B_ = 8192
D_IN_ = 512
HID_ = 64
COMM_ = 256
K_ = 8192
BETA_ = 0.25
KL_WEIGHT_ = 1.0

TB = 512
KT = 1024
NBT = B_ // TB
NKT = K_ // KT


def _fused_body(x_ref, we, be, w0, b0, w1, b1, wmu, bmu, wvar, bvar,
                pt_ref, eps_ref, closest_ref, mu_ref, kld_ref, pn_ref):
    i = pl.program_id(0)

    @pl.when(i == 0)
    def _():
        kld_ref[0, 0] = 0.0
        for k in range(NKT):
            pt = pt_ref[:, k * KT:(k + 1) * KT]
            pn_ref[:, k * KT:(k + 1) * KT] = jnp.sum(
                pt ** 2, axis=0, keepdims=True)

    x = x_ref[...]
    emb = jnp.dot(x, we[...]) + be[...]
    h = jnp.maximum(jnp.dot(emb, w0[...]) + b0[...], 0.0)
    h = jnp.maximum(jnp.dot(h, w1[...]) + b1[...], 0.0)
    mu = jnp.dot(h, wmu[...]) + bmu[...]
    lv = jnp.dot(h, wvar[...]) + bvar[...]
    sample = mu + eps_ref[...] * jnp.exp(0.5 * lv)
    mu_ref[...] = mu

    snorm = jnp.sum(sample ** 2, axis=1, keepdims=True)
    sample2 = sample + sample

    iotaf = lax.broadcasted_iota(jnp.int32, (1, KT), 1).astype(jnp.float32)
    rmin = jnp.full((TB, 1), jnp.inf, jnp.float32)
    ridxf = jnp.zeros((TB, 1), jnp.float32)
    for k in range(NKT):
        pt = pt_ref[:, k * KT:(k + 1) * KT]
        pn = pn_ref[:, k * KT:(k + 1) * KT]
        m2 = jnp.dot(sample2, pt)
        d = snorm + pn - m2
        cmin = jnp.min(d, axis=1, keepdims=True)
        cidxf = jnp.min(jnp.where(d == cmin, iotaf + (k * KT), float(K_)),
                        axis=1, keepdims=True)
        better = cmin < rmin
        rmin = jnp.where(better, cmin, rmin)
        ridxf = jnp.where(better, cidxf, ridxf)
    closest_ref[...] = ridxf.astype(jnp.int32)

    kld_ref[0, 0] += jnp.sum((1.0 + lv) - mu ** 2 - jnp.exp(lv))


def _loss_body(q_ref, mu_ref, kldsum_ref, total_ref, cap_ref):
    i = pl.program_id(0)

    @pl.when(i == 0)
    def _():
        total_ref[0, 0] = 0.0

    diff = q_ref[...] - mu_ref[...]
    total_ref[0, 0] += jnp.sum(diff * diff)

    @pl.when(i == NBT - 1)
    def _():
        msq = total_ref[0, 0] / (B_ * COMM_)
        vq = 1.0 * (msq * BETA_ + 1.0 * msq)
        kld = -0.5 * kldsum_ref[0, 0] / B_
        total_ref[0, 0] = KL_WEIGHT_ * kld + vq
        cap_ref[0, 0] = kld


@functools.cache
def _make_sc_gather():
    info = plsc.get_sparse_core_info()
    nc, ns = info.num_cores, info.num_subcores
    bpw = B_ // (nc * ns)

    @functools.partial(
        pl.kernel,
        out_type=jax.ShapeDtypeStruct((B_, COMM_), jnp.float32),
        mesh=plsc.VectorSubcoreMesh(core_axis_name="c", subcore_axis_name="s"),
        scratch_types=[
            pltpu.VMEM((bpw,), jnp.int32),
            pltpu.VMEM((bpw, COMM_), jnp.float32),
            pltpu.SemaphoreType.DMA,
        ],
    )
    def _sc_gather(table_hbm, idx_hbm, out_hbm, idx_v, rows_v, sem):
        wid = lax.axis_index("s") * nc + lax.axis_index("c")
        base = wid * bpw
        pltpu.sync_copy(idx_hbm.at[pl.ds(base, bpw)], idx_v)
        pltpu.async_copy(table_hbm.at[idx_v], rows_v, sem).wait()
        pltpu.sync_copy(rows_v, out_hbm.at[pl.ds(base, bpw)])

    return _sc_gather


def kernel(x, W_emb, b_emb, W0, b0, W1, b1, W_mu, b_mu, W_var, b_var,
           protos, eps):
    pt = protos.T
    b_emb2 = b_emb.reshape(1, HID_)
    b02 = b0.reshape(1, HID_)
    b12 = b1.reshape(1, COMM_)
    b_mu2 = b_mu.reshape(1, COMM_)
    b_var2 = b_var.reshape(1, COMM_)

    def full(shape):
        return pl.BlockSpec(shape, lambda i: (0, 0))

    closest2d, mu, kld_sum = pl.pallas_call(
        _fused_body,
        grid=(NBT,),
        in_specs=[
            pl.BlockSpec((TB, D_IN_), lambda i: (i, 0)),
            full((D_IN_, HID_)),
            full((1, HID_)),
            full((HID_, HID_)),
            full((1, HID_)),
            full((HID_, COMM_)),
            full((1, COMM_)),
            full((COMM_, COMM_)),
            full((1, COMM_)),
            full((COMM_, COMM_)),
            full((1, COMM_)),
            full((COMM_, K_)),
            pl.BlockSpec((TB, COMM_), lambda i: (i, 0)),
        ],
        out_specs=[
            pl.BlockSpec((TB, 1), lambda i: (i, 0)),
            pl.BlockSpec((TB, COMM_), lambda i: (i, 0)),
            pl.BlockSpec(memory_space=pltpu.SMEM),
        ],
        out_shape=[
            jax.ShapeDtypeStruct((B_, 1), jnp.int32),
            jax.ShapeDtypeStruct((B_, COMM_), jnp.float32),
            jax.ShapeDtypeStruct((1, 1), jnp.float32),
        ],
        scratch_shapes=[pltpu.VMEM((1, K_), jnp.float32)],
    )(x, W_emb, b_emb2, W0, b02, W1, b12, W_mu, b_mu2, W_var, b_var2,
      pt, eps)

    closest = closest2d.reshape(B_)
    quantized = _make_sc_gather()(protos, closest)

    total, cap = pl.pallas_call(
        _loss_body,
        grid=(NBT,),
        in_specs=[
            pl.BlockSpec((TB, COMM_), lambda i: (i, 0)),
            pl.BlockSpec((TB, COMM_), lambda i: (i, 0)),
            pl.BlockSpec(memory_space=pltpu.SMEM),
        ],
        out_specs=[
            pl.BlockSpec(memory_space=pltpu.SMEM),
            pl.BlockSpec(memory_space=pltpu.SMEM),
        ],
        out_shape=[
            jax.ShapeDtypeStruct((1, 1), jnp.float32),
            jax.ShapeDtypeStruct((1, 1), jnp.float32),
        ],
    )(quantized, mu, kld_sum)

    return (quantized, total.reshape(()), cap.reshape(()))


# speedup vs baseline: 3.6729x; 1.0606x over previous
import functools

import jax
import jax.numpy as jnp
from jax import lax
from jax.experimental import pallas as pl
from jax.experimental.pallas import tpu as pltpu
from jax.experimental.pallas import tpu_sc as plsc

B_ = 8192
D_IN_ = 512
HID_ = 64
COMM_ = 256
K_ = 8192
BETA_ = 0.25
KL_WEIGHT_ = 1.0

TB = 512
KT = 1024
NBT = B_ // TB
NKT = K_ // KT


_DN_T = (((1,), (1,)), ((), ()))


def _fused_body(x_ref, we, be, w0, b0, w1, b1, wmu, bmu, wvar, bvar,
                p_ref, eps_ref, closest_ref, mu_ref, kld_ref, pn_ref):
    i = pl.program_id(0)

    @pl.when(i == 0)
    def _():
        kld_ref[0, 0] = 0.0
        ones = jnp.ones((8, COMM_), jnp.float32)
        for k in range(NKT):
            pk = p_ref[k * KT:(k + 1) * KT, :]
            pn_ref[:, k * KT:(k + 1) * KT] = lax.dot_general(
                ones, pk * pk, _DN_T)[:1]

    x = x_ref[...]
    emb = jnp.dot(x, we[...]) + be[...]
    h = jnp.maximum(jnp.dot(emb, w0[...]) + b0[...], 0.0)
    h = jnp.maximum(jnp.dot(h, w1[...]) + b1[...], 0.0)
    mu = jnp.dot(h, wmu[...]) + bmu[...]
    lv = jnp.dot(h, wvar[...]) + bvar[...]
    sample = mu + eps_ref[...] * jnp.exp(0.5 * lv)
    mu_ref[...] = mu

    snorm = jnp.sum(sample ** 2, axis=1, keepdims=True)
    sample2 = sample + sample

    iotaf = lax.broadcasted_iota(jnp.int32, (1, KT), 1).astype(jnp.float32)
    rmin = jnp.full((TB, 1), jnp.inf, jnp.float32)
    ridxf = jnp.zeros((TB, 1), jnp.float32)
    for k in range(NKT):
        pk = p_ref[k * KT:(k + 1) * KT, :]
        pn = pn_ref[:, k * KT:(k + 1) * KT]
        m2 = lax.dot_general(sample2, pk, _DN_T)
        d = snorm + pn - m2
        cmin = jnp.min(d, axis=1, keepdims=True)
        cidxf = jnp.min(jnp.where(d == cmin, iotaf + (k * KT), float(K_)),
                        axis=1, keepdims=True)
        better = cmin < rmin
        rmin = jnp.where(better, cmin, rmin)
        ridxf = jnp.where(better, cidxf, ridxf)
    closest_ref[...] = ridxf.astype(jnp.int32)

    kld_ref[0, 0] += jnp.sum((1.0 + lv) - mu ** 2 - jnp.exp(lv))


def _loss_body(q_ref, mu_ref, kldsum_ref, total_ref, cap_ref):
    i = pl.program_id(0)

    @pl.when(i == 0)
    def _():
        total_ref[0, 0] = 0.0

    diff = q_ref[...] - mu_ref[...]
    total_ref[0, 0] += jnp.sum(diff * diff)

    @pl.when(i == NBT - 1)
    def _():
        msq = total_ref[0, 0] / (B_ * COMM_)
        vq = 1.0 * (msq * BETA_ + 1.0 * msq)
        kld = -0.5 * kldsum_ref[0, 0] / B_
        total_ref[0, 0] = KL_WEIGHT_ * kld + vq
        cap_ref[0, 0] = kld


@functools.cache
def _make_sc_gather():
    info = plsc.get_sparse_core_info()
    nc, ns = info.num_cores, info.num_subcores
    bpw = B_ // (nc * ns)

    @functools.partial(
        pl.kernel,
        out_type=jax.ShapeDtypeStruct((B_, COMM_), jnp.float32),
        mesh=plsc.VectorSubcoreMesh(core_axis_name="c", subcore_axis_name="s"),
        scratch_types=[
            pltpu.VMEM((bpw,), jnp.int32),
            pltpu.VMEM((bpw, COMM_), jnp.float32),
            pltpu.SemaphoreType.DMA,
        ],
    )
    def _sc_gather(table_hbm, idx_hbm, out_hbm, idx_v, rows_v, sem):
        wid = lax.axis_index("s") * nc + lax.axis_index("c")
        base = wid * bpw
        pltpu.sync_copy(idx_hbm.at[pl.ds(base, bpw)], idx_v)
        pltpu.async_copy(table_hbm.at[idx_v], rows_v, sem).wait()
        pltpu.sync_copy(rows_v, out_hbm.at[pl.ds(base, bpw)])

    return _sc_gather


def kernel(x, W_emb, b_emb, W0, b0, W1, b1, W_mu, b_mu, W_var, b_var,
           protos, eps):
    b_emb2 = b_emb.reshape(1, HID_)
    b02 = b0.reshape(1, HID_)
    b12 = b1.reshape(1, COMM_)
    b_mu2 = b_mu.reshape(1, COMM_)
    b_var2 = b_var.reshape(1, COMM_)

    def full(shape):
        return pl.BlockSpec(shape, lambda i: (0, 0))

    closest2d, mu, kld_sum = pl.pallas_call(
        _fused_body,
        grid=(NBT,),
        in_specs=[
            pl.BlockSpec((TB, D_IN_), lambda i: (i, 0)),
            full((D_IN_, HID_)),
            full((1, HID_)),
            full((HID_, HID_)),
            full((1, HID_)),
            full((HID_, COMM_)),
            full((1, COMM_)),
            full((COMM_, COMM_)),
            full((1, COMM_)),
            full((COMM_, COMM_)),
            full((1, COMM_)),
            full((K_, COMM_)),
            pl.BlockSpec((TB, COMM_), lambda i: (i, 0)),
        ],
        out_specs=[
            pl.BlockSpec((TB, 1), lambda i: (i, 0)),
            pl.BlockSpec((TB, COMM_), lambda i: (i, 0)),
            pl.BlockSpec(memory_space=pltpu.SMEM),
        ],
        out_shape=[
            jax.ShapeDtypeStruct((B_, 1), jnp.int32),
            jax.ShapeDtypeStruct((B_, COMM_), jnp.float32),
            jax.ShapeDtypeStruct((1, 1), jnp.float32),
        ],
        scratch_shapes=[pltpu.VMEM((1, K_), jnp.float32)],
    )(x, W_emb, b_emb2, W0, b02, W1, b12, W_mu, b_mu2, W_var, b_var2,
      protos, eps)

    quantized = _make_sc_gather()(protos, closest2d.reshape(B_))

    total, cap = pl.pallas_call(
        _loss_body,
        grid=(NBT,),
        in_specs=[
            pl.BlockSpec((TB, COMM_), lambda i: (i, 0)),
            pl.BlockSpec((TB, COMM_), lambda i: (i, 0)),
            pl.BlockSpec(memory_space=pltpu.SMEM),
        ],
        out_specs=[
            pl.BlockSpec(memory_space=pltpu.SMEM),
            pl.BlockSpec(memory_space=pltpu.SMEM),
        ],
        out_shape=[
            jax.ShapeDtypeStruct((1, 1), jnp.float32),
            jax.ShapeDtypeStruct((1, 1), jnp.float32),
        ],
    )(quantized, mu, kld_sum)

    return (quantized, total.reshape(()), cap.reshape(()))


# speedup vs baseline: 4.1260x; 1.1234x over previous
import functools

import jax
import jax.numpy as jnp
from jax import lax
from jax.experimental import pallas as pl
from jax.experimental.pallas import tpu as pltpu
from jax.experimental.pallas import tpu_sc as plsc

B_ = 8192
D_IN_ = 512
HID_ = 64
COMM_ = 256
K_ = 8192
BETA_ = 0.25
KL_WEIGHT_ = 1.0

TB = 2048
KT = 1024
NBT = B_ // TB
NKT = K_ // KT


_DN_T = (((1,), (1,)), ((), ()))


def _fused_body(x_ref, we, be, w0, b0, w1, b1, wmu, bmu, wvar, bvar,
                p_ref, eps_ref, closest_ref, mu_ref, kld_ref, pn_ref):
    i = pl.program_id(0)

    @pl.when(i == 0)
    def _():
        kld_ref[0, 0] = 0.0
        ones = jnp.ones((8, COMM_), jnp.float32)
        for k in range(NKT):
            pk = p_ref[k * KT:(k + 1) * KT, :]
            pn_ref[:, k * KT:(k + 1) * KT] = lax.dot_general(
                ones, pk * pk, _DN_T)[:1]

    x = x_ref[...]
    emb = jnp.dot(x, we[...]) + be[...]
    h = jnp.maximum(jnp.dot(emb, w0[...]) + b0[...], 0.0)
    h = jnp.maximum(jnp.dot(h, w1[...]) + b1[...], 0.0)
    mu = jnp.dot(h, wmu[...]) + bmu[...]
    lv = jnp.dot(h, wvar[...]) + bvar[...]
    sample = mu + eps_ref[...] * jnp.exp(0.5 * lv)
    mu_ref[...] = mu

    snorm = jnp.sum(sample ** 2, axis=1, keepdims=True)
    sample2 = sample + sample

    iotaf = lax.broadcasted_iota(jnp.int32, (1, KT), 1).astype(jnp.float32)
    rmin = jnp.full((TB, 1), jnp.inf, jnp.float32)
    ridxf = jnp.zeros((TB, 1), jnp.float32)
    for k in range(NKT):
        pk = p_ref[k * KT:(k + 1) * KT, :]
        pn = pn_ref[:, k * KT:(k + 1) * KT]
        m2 = lax.dot_general(sample2, pk, _DN_T)
        d = snorm + pn - m2
        cmin = jnp.min(d, axis=1, keepdims=True)
        cidxf = jnp.min(jnp.where(d == cmin, iotaf + (k * KT), float(K_)),
                        axis=1, keepdims=True)
        better = cmin < rmin
        rmin = jnp.where(better, cmin, rmin)
        ridxf = jnp.where(better, cidxf, ridxf)
    closest_ref[...] = ridxf.astype(jnp.int32)

    kld_ref[0, 0] += jnp.sum((1.0 + lv) - mu ** 2 - jnp.exp(lv))


def _loss_body(q_ref, mu_ref, kldsum_ref, total_ref, cap_ref):
    i = pl.program_id(0)

    @pl.when(i == 0)
    def _():
        total_ref[0, 0] = 0.0

    diff = q_ref[...] - mu_ref[...]
    total_ref[0, 0] += jnp.sum(diff * diff)

    @pl.when(i == NBT - 1)
    def _():
        msq = total_ref[0, 0] / (B_ * COMM_)
        vq = 1.0 * (msq * BETA_ + 1.0 * msq)
        kld = -0.5 * kldsum_ref[0, 0] / B_
        total_ref[0, 0] = KL_WEIGHT_ * kld + vq
        cap_ref[0, 0] = kld


@functools.cache
def _make_sc_gather():
    info = plsc.get_sparse_core_info()
    nc, ns = info.num_cores, info.num_subcores
    bpw = B_ // (nc * ns)

    @functools.partial(
        pl.kernel,
        out_type=jax.ShapeDtypeStruct((B_, COMM_), jnp.float32),
        mesh=plsc.VectorSubcoreMesh(core_axis_name="c", subcore_axis_name="s"),
        scratch_types=[
            pltpu.VMEM((bpw,), jnp.int32),
            pltpu.VMEM((bpw, COMM_), jnp.float32),
            pltpu.SemaphoreType.DMA,
        ],
    )
    def _sc_gather(table_hbm, idx_hbm, out_hbm, idx_v, rows_v, sem):
        wid = lax.axis_index("s") * nc + lax.axis_index("c")
        base = wid * bpw
        pltpu.sync_copy(idx_hbm.at[pl.ds(base, bpw)], idx_v)
        pltpu.async_copy(table_hbm.at[idx_v], rows_v, sem).wait()
        pltpu.sync_copy(rows_v, out_hbm.at[pl.ds(base, bpw)])

    return _sc_gather


def kernel(x, W_emb, b_emb, W0, b0, W1, b1, W_mu, b_mu, W_var, b_var,
           protos, eps):
    b_emb2 = b_emb.reshape(1, HID_)
    b02 = b0.reshape(1, HID_)
    b12 = b1.reshape(1, COMM_)
    b_mu2 = b_mu.reshape(1, COMM_)
    b_var2 = b_var.reshape(1, COMM_)

    def full(shape):
        return pl.BlockSpec(shape, lambda i: (0, 0))

    closest2d, mu, kld_sum = pl.pallas_call(
        _fused_body,
        grid=(NBT,),
        in_specs=[
            pl.BlockSpec((TB, D_IN_), lambda i: (i, 0)),
            full((D_IN_, HID_)),
            full((1, HID_)),
            full((HID_, HID_)),
            full((1, HID_)),
            full((HID_, COMM_)),
            full((1, COMM_)),
            full((COMM_, COMM_)),
            full((1, COMM_)),
            full((COMM_, COMM_)),
            full((1, COMM_)),
            full((K_, COMM_)),
            pl.BlockSpec((TB, COMM_), lambda i: (i, 0)),
        ],
        out_specs=[
            pl.BlockSpec((TB, 1), lambda i: (i, 0)),
            pl.BlockSpec((TB, COMM_), lambda i: (i, 0)),
            pl.BlockSpec(memory_space=pltpu.SMEM),
        ],
        out_shape=[
            jax.ShapeDtypeStruct((B_, 1), jnp.int32),
            jax.ShapeDtypeStruct((B_, COMM_), jnp.float32),
            jax.ShapeDtypeStruct((1, 1), jnp.float32),
        ],
        scratch_shapes=[pltpu.VMEM((1, K_), jnp.float32)],
    )(x, W_emb, b_emb2, W0, b02, W1, b12, W_mu, b_mu2, W_var, b_var2,
      protos, eps)

    quantized = _make_sc_gather()(protos, closest2d.reshape(B_))

    total, cap = pl.pallas_call(
        _loss_body,
        grid=(NBT,),
        in_specs=[
            pl.BlockSpec((TB, COMM_), lambda i: (i, 0)),
            pl.BlockSpec((TB, COMM_), lambda i: (i, 0)),
            pl.BlockSpec(memory_space=pltpu.SMEM),
        ],
        out_specs=[
            pl.BlockSpec(memory_space=pltpu.SMEM),
            pl.BlockSpec(memory_space=pltpu.SMEM),
        ],
        out_shape=[
            jax.ShapeDtypeStruct((1, 1), jnp.float32),
            jax.ShapeDtypeStruct((1, 1), jnp.float32),
        ],
    )(quantized, mu, kld_sum)

    return (quantized, total.reshape(()), cap.reshape(()))
